# baseline (speedup 1.0000x reference)
import functools

import jax
import jax.numpy as jnp
from jax import lax
from jax.experimental import pallas as pl
from jax.experimental.pallas import tpu as pltpu
from jax.experimental.pallas import tpu_sc as plsc

N = 10000
E = 160000
D = 256
H = 256
G = 8

NPAD = 10240
EPAD = 163840
NW = 32
LROWS = NPAD // NW
LACC = LROWS + 16
BS = 128
SEG = 8192
NSEG = EPAD // SEG
QPS = SEG // 16
SEGBUF = SEG + 256

HBS = BS // 2

ROW_BLK = 1024
NROW_BLK = NPAD // ROW_BLK

_MESH = plsc.VectorSubcoreMesh(core_axis_name="c", subcore_axis_name="s")


@functools.partial(
    pl.kernel,
    out_type=(
        jax.ShapeDtypeStruct((NPAD,), jnp.float32),
        jax.ShapeDtypeStruct((NW, 16), jnp.int32),
        jax.ShapeDtypeStruct((NW, EPAD), jnp.int32),
        jax.ShapeDtypeStruct((NW, EPAD), jnp.int32),
    ),
    mesh=_MESH,
    compiler_params=pltpu.CompilerParams(needs_layout_passes=False),
    scratch_types=[
        pltpu.VMEM((SEG,), jnp.int32),
        pltpu.VMEM((SEG,), jnp.int32),
        pltpu.VMEM((SEGBUF,), jnp.int32),
        pltpu.VMEM((SEGBUF,), jnp.int32),
        pltpu.VMEM((16 * LACC,), jnp.float32),
        pltpu.VMEM((LROWS,), jnp.float32),
        pltpu.VMEM((16,), jnp.int32),
    ],
)
def _prep_kernel(didx_hbm, src_hbm, deg_hbm, cnt_hbm, lsrc_hbm, lld_hbm,
                 dv, sv, bsrc, bld, hist, degv, cntv):
    c = lax.axis_index("c")
    s = lax.axis_index("s")
    w = s * 2 + c
    lo = w * LROWS

    @pl.loop(0, LACC)
    def _(i):
        hist[pl.ds(i * 16, 16)] = jnp.zeros((16,), jnp.float32)

    lane = lax.iota(jnp.int32, 16)
    ones = jnp.ones((16,), jnp.float32)

    def scan_segment(seg, carry):
        cnt, blk = carry
        pltpu.sync_copy(didx_hbm.at[pl.ds(seg * SEG, SEG)], dv)
        pltpu.sync_copy(src_hbm.at[pl.ds(seg * SEG, SEG)], sv)

        def step(q, cnt):
            d = dv[pl.ds(q * 16, 16)]
            s16 = sv[pl.ds(q * 16, 16)]
            local = d - lo
            m = (local >= 0) & (local < LROWS)
            lidx = jnp.clip(local, 0, LROWS - 1)
            plsc.addupdate_scatter(hist, [lane * LACC + lidx], ones, mask=m)
            plsc.store_compressed(bsrc.at[pl.ds(cnt, 16)], s16, mask=m)
            plsc.store_compressed(bld.at[pl.ds(cnt, 16)], lidx, mask=m)
            return cnt + plsc.all_reduce_population_count(m)[0]

        cnt = lax.fori_loop(0, QPS, step, cnt)
        nfull = cnt // BS

        def flush(b, _):
            pltpu.sync_copy(bsrc.at[pl.ds(b * BS, BS)],
                            lsrc_hbm.at[w, pl.ds((blk + b) * BS, BS)])
            pltpu.sync_copy(bld.at[pl.ds(b * BS, BS)],
                            lld_hbm.at[w, pl.ds((blk + b) * BS, BS)])
            return 0

        lax.fori_loop(0, nfull, flush, 0)
        rem = cnt - nfull * BS

        @pl.loop(0, 8)
        def _(t):
            bsrc[pl.ds(t * 16, 16)] = bsrc[pl.ds(nfull * BS + t * 16, 16)]
            bld[pl.ds(t * 16, 16)] = bld[pl.ds(nfull * BS + t * 16, 16)]

        return rem, blk + nfull

    cnt, blk = lax.fori_loop(0, NSEG, scan_segment, (0, 0))

    @pl.loop(0, 8)
    def _(t):
        bsrc[pl.ds(cnt + t * 16, 16)] = lane * 64 + w * 17 + t
        bld[pl.ds(cnt + t * 16, 16)] = jnp.full((16,), LROWS, jnp.int32) + lane

    cnt = cnt + BS
    nfull = cnt // BS

    def flush2(b, _):
        pltpu.sync_copy(bsrc.at[pl.ds(b * BS, BS)],
                        lsrc_hbm.at[w, pl.ds((blk + b) * BS, BS)])
        pltpu.sync_copy(bld.at[pl.ds(b * BS, BS)],
                        lld_hbm.at[w, pl.ds((blk + b) * BS, BS)])
        return 0

    lax.fori_loop(0, nfull, flush2, 0)
    blk = blk + nfull

    cntv[...] = jnp.full((16,), 0, jnp.int32) + blk
    pltpu.sync_copy(cntv, cnt_hbm.at[w])

    @pl.loop(0, LROWS // 16)
    def _(i):
        acc = jnp.zeros((16,), jnp.float32)
        for j in range(16):
            acc = acc + hist[pl.ds(j * LACC + i * 16, 16)]
        degv[pl.ds(i * 16, 16)] = acc

    pltpu.sync_copy(degv, deg_hbm.at[pl.ds(lo, LROWS)])


@functools.partial(
    pl.kernel,
    out_type=jax.ShapeDtypeStruct((NPAD, H), jnp.float32),
    mesh=_MESH,
    compiler_params=pltpu.CompilerParams(needs_layout_passes=False),
    scratch_types=[
        pltpu.VMEM((2, HBS), jnp.int32),
        pltpu.VMEM((BS,), jnp.int32),
        pltpu.VMEM((16,), jnp.int32),
        pltpu.VMEM((2, HBS, H), jnp.float32),
        pltpu.VMEM((LACC, H), jnp.float32),
        pltpu.SemaphoreType.DMA,
        pltpu.SemaphoreType.DMA,
    ],
)
def _agg_kernel(lsrc_hbm, lld_hbm, cnt_hbm, y_hbm, out_hbm,
                sidx_v, ldv_v, cntv_v, rows_v, acc_v, sem_a, sem_b):
    c = lax.axis_index("c")
    s = lax.axis_index("s")
    w = s * 2 + c
    row0 = w * LROWS
    pltpu.sync_copy(y_hbm.at[pl.ds(row0, LROWS)], acc_v.at[pl.ds(0, LROWS)])
    pltpu.sync_copy(cnt_hbm.at[w], cntv_v)
    nblk = cntv_v[pl.ds(0, 16)][0]
    sems = (sem_a, sem_b)

    def start(b64, buf):
        pltpu.sync_copy(lsrc_hbm.at[w, pl.ds(b64 * HBS, HBS)],
                        sidx_v.at[buf])
        pltpu.async_copy(y_hbm.at[sidx_v.at[buf]], rows_v.at[buf], sems[buf])

    def finish(buf):
        pltpu.make_async_copy(y_hbm.at[sidx_v.at[buf]], rows_v.at[buf],
                              sems[buf]).wait()

    def accum(buf, half):
        for q in range(HBS // 16):
            ld16 = ldv_v[pl.ds(half * HBS + q * 16, 16)]
            lds = [ld16[j] for j in range(16)]
            for k in range(H // 16):
                sl = pl.ds(k * 16, 16)
                for j in range(16):
                    plsc.addupdate(acc_v.at[lds[j], sl],
                                   rows_v[buf, q * 16 + j, sl])

    start(0, 0)

    @pl.loop(0, nblk)
    def _(b):
        pltpu.sync_copy(lld_hbm.at[w, pl.ds(b * BS, BS)], ldv_v)
        start(b * 2 + 1, 1)
        finish(0)
        accum(0, 0)

        @pl.when(b + 1 < nblk)
        def _():
            start(b * 2 + 2, 0)

        finish(1)
        accum(1, 1)

    pltpu.sync_copy(acc_v.at[pl.ds(0, LROWS)], out_hbm.at[pl.ds(row0, LROWS)])


def _dinv_col(deg_ref):
    return 1.0 / jnp.sqrt(deg_ref[...] + 1.0)


def _tc1_body(deg_ref, x_ref, w_ref, o_ref):
    dinv = _dinv_col(deg_ref)
    o_ref[...] = jnp.dot(x_ref[...], w_ref[...],
                         preferred_element_type=jnp.float32) * dinv


def _tc2_body(deg_ref, agg_ref, b_ref, w_ref, o_ref):
    dinv = _dinv_col(deg_ref)
    h = jnp.maximum(agg_ref[...] * dinv + b_ref[...], 0.0)
    o_ref[...] = jnp.dot(h, w_ref[...],
                         preferred_element_type=jnp.float32) * dinv


def _tc3_body(deg_ref, agg_ref, b_ref, batch_ref, wh_ref, bh_ref, o_ref,
              sums, cnts):
    i = pl.program_id(0)

    @pl.when(i == 0)
    def _():
        sums[...] = jnp.zeros_like(sums)
        cnts[...] = jnp.zeros_like(cnts)

    dinv = _dinv_col(deg_ref)
    h = jnp.maximum(agg_ref[...] * dinv + b_ref[...], 0.0)
    oh = (batch_ref[...] ==
          lax.broadcasted_iota(jnp.int32, (G, ROW_BLK), 0)).astype(jnp.float32)
    sums[...] += jnp.dot(oh, h, preferred_element_type=jnp.float32,
                         precision=lax.Precision.HIGHEST)
    cnts[...] += jnp.dot(oh, jnp.ones((ROW_BLK, H), jnp.float32),
                         preferred_element_type=jnp.float32,
                         precision=lax.Precision.HIGHEST)

    @pl.when(i == NROW_BLK - 1)
    def _():
        pooled = sums[...] / jnp.maximum(cnts[...], 1.0)
        o_ref[...] = jnp.dot(pooled, wh_ref[...],
                             preferred_element_type=jnp.float32,
                         precision=lax.Precision.HIGHEST) + bh_ref[...]


def _tc1(deg2d, x_pad, W):
    return pl.pallas_call(
        _tc1_body,
        grid=(NROW_BLK,),
        in_specs=[
            pl.BlockSpec((ROW_BLK, 1), lambda i: (i, 0)),
            pl.BlockSpec((ROW_BLK, D), lambda i: (i, 0)),
            pl.BlockSpec((D, H), lambda i: (0, 0)),
        ],
        out_specs=pl.BlockSpec((ROW_BLK, H), lambda i: (i, 0)),
        out_shape=jax.ShapeDtypeStruct((NPAD, H), jnp.float32),
    )(deg2d, x_pad, W)


def _tc2(deg2d, agg, b2d, W):
    return pl.pallas_call(
        _tc2_body,
        grid=(NROW_BLK,),
        in_specs=[
            pl.BlockSpec((ROW_BLK, 1), lambda i: (i, 0)),
            pl.BlockSpec((ROW_BLK, H), lambda i: (i, 0)),
            pl.BlockSpec((1, H), lambda i: (0, 0)),
            pl.BlockSpec((H, H), lambda i: (0, 0)),
        ],
        out_specs=pl.BlockSpec((ROW_BLK, H), lambda i: (i, 0)),
        out_shape=jax.ShapeDtypeStruct((NPAD, H), jnp.float32),
    )(deg2d, agg, b2d, W)


def _tc3(deg2d, agg, b2d, batch2d, Wh, bh2d):
    return pl.pallas_call(
        _tc3_body,
        grid=(NROW_BLK,),
        in_specs=[
            pl.BlockSpec((ROW_BLK, 1), lambda i: (i, 0)),
            pl.BlockSpec((ROW_BLK, H), lambda i: (i, 0)),
            pl.BlockSpec((1, H), lambda i: (0, 0)),
            pl.BlockSpec((1, ROW_BLK), lambda i: (0, i)),
            pl.BlockSpec((H, 1), lambda i: (0, 0)),
            pl.BlockSpec((1, 1), lambda i: (0, 0)),
        ],
        out_specs=pl.BlockSpec((G, 1), lambda i: (0, 0)),
        out_shape=jax.ShapeDtypeStruct((G, 1), jnp.float32),
        scratch_shapes=[
            pltpu.VMEM((G, H), jnp.float32),
            pltpu.VMEM((G, H), jnp.float32),
        ],
    )(deg2d, agg, b2d, batch2d, Wh, bh2d)


def kernel(x, edge_index, batch, W1, b1, W2, b2, Wh, bh):
    src = edge_index[0].astype(jnp.int32)
    dst = edge_index[1].astype(jnp.int32)
    pad_e = EPAD - E

    pad_ids = jnp.arange(pad_e, dtype=jnp.int32)
    src_pad = jnp.concatenate([src, (pad_ids * 97) % N])
    didx = jnp.concatenate([dst, N + pad_ids % (NPAD - N)])

    x_pad = jnp.pad(x, ((0, NPAD - N), (0, 0)))
    batch2d = jnp.pad(batch.astype(jnp.int32), (0, NPAD - N),
                      constant_values=G).reshape(1, NPAD)
    b1_2d = b1.reshape(1, H)
    b2_2d = b2.reshape(1, H)
    bh2d = bh.reshape(1, 1)

    deg, cnt16, lsrc, lld = _prep_kernel(didx, src_pad)
    deg2d = deg.reshape(NPAD, 1)

    y1 = _tc1(deg2d, x_pad, W1)
    agg1 = _agg_kernel(lsrc, lld, cnt16, y1)
    y2 = _tc2(deg2d, agg1, b1_2d, W2)
    agg2 = _agg_kernel(lsrc, lld, cnt16, y2)
    return _tc3(deg2d, agg2, b2_2d, batch2d, Wh, bh2d)


# speedup vs baseline: 1.2826x; 1.2826x over previous
import functools

import jax
import jax.numpy as jnp
from jax import lax
from jax.experimental import pallas as pl
from jax.experimental.pallas import tpu as pltpu
from jax.experimental.pallas import tpu_sc as plsc

N = 10000
E = 160000
D = 256
H = 256
G = 8

NPAD = 10240
EPAD = 163840
NW = 32
LROWS = NPAD // NW
LACC = LROWS + 16
BS = 128
SEG = 8192
NSEG = EPAD // SEG
QPS = SEG // 16
SEGBUF = SEG + 256

HBS = BS // 2

ROW_BLK = 1024
NROW_BLK = NPAD // ROW_BLK

_MESH = plsc.VectorSubcoreMesh(core_axis_name="c", subcore_axis_name="s")


@functools.partial(
    pl.kernel,
    out_type=(
        jax.ShapeDtypeStruct((NPAD,), jnp.float32),
        jax.ShapeDtypeStruct((NW, 16), jnp.int32),
        jax.ShapeDtypeStruct((NW, EPAD), jnp.int32),
        jax.ShapeDtypeStruct((NW, EPAD), jnp.int32),
    ),
    mesh=_MESH,
    compiler_params=pltpu.CompilerParams(needs_layout_passes=False),
    scratch_types=[
        pltpu.VMEM((SEG,), jnp.int32),
        pltpu.VMEM((SEG,), jnp.int32),
        pltpu.VMEM((SEGBUF,), jnp.int32),
        pltpu.VMEM((SEGBUF,), jnp.int32),
        pltpu.VMEM((16 * LACC,), jnp.float32),
        pltpu.VMEM((LROWS,), jnp.float32),
        pltpu.VMEM((16,), jnp.int32),
    ],
)
def _prep_kernel(didx_hbm, src_hbm, deg_hbm, cnt_hbm, lsrc_hbm, lld_hbm,
                 dv, sv, bsrc, bld, hist, degv, cntv):
    c = lax.axis_index("c")
    s = lax.axis_index("s")
    w = s * 2 + c
    lo = w * LROWS

    @pl.loop(0, LACC)
    def _(i):
        hist[pl.ds(i * 16, 16)] = jnp.zeros((16,), jnp.float32)

    lane = lax.iota(jnp.int32, 16)
    ones = jnp.ones((16,), jnp.float32)

    def scan_segment(seg, carry):
        cnt, blk = carry
        pltpu.sync_copy(didx_hbm.at[pl.ds(seg * SEG, SEG)], dv)
        pltpu.sync_copy(src_hbm.at[pl.ds(seg * SEG, SEG)], sv)

        def step(q, cnt):
            d = dv[pl.ds(q * 16, 16)]
            s16 = sv[pl.ds(q * 16, 16)]
            local = d - lo
            m = (local >= 0) & (local < LROWS)
            lidx = jnp.clip(local, 0, LROWS - 1)
            plsc.addupdate_scatter(hist, [lane * LACC + lidx], ones, mask=m)
            plsc.store_compressed(bsrc.at[pl.ds(cnt, 16)], s16, mask=m)
            plsc.store_compressed(bld.at[pl.ds(cnt, 16)], lidx, mask=m)
            return cnt + plsc.all_reduce_population_count(m)[0]

        cnt = lax.fori_loop(0, QPS, step, cnt)
        nfull = cnt // BS

        def flush(b, _):
            pltpu.sync_copy(bsrc.at[pl.ds(b * BS, BS)],
                            lsrc_hbm.at[w, pl.ds((blk + b) * BS, BS)])
            pltpu.sync_copy(bld.at[pl.ds(b * BS, BS)],
                            lld_hbm.at[w, pl.ds((blk + b) * BS, BS)])
            return 0

        lax.fori_loop(0, nfull, flush, 0)
        rem = cnt - nfull * BS

        @pl.loop(0, 8)
        def _(t):
            bsrc[pl.ds(t * 16, 16)] = bsrc[pl.ds(nfull * BS + t * 16, 16)]
            bld[pl.ds(t * 16, 16)] = bld[pl.ds(nfull * BS + t * 16, 16)]

        return rem, blk + nfull

    cnt, blk = lax.fori_loop(0, NSEG, scan_segment, (0, 0))

    @pl.loop(0, 8)
    def _(t):
        bsrc[pl.ds(cnt + t * 16, 16)] = lane * 64 + w * 17 + t
        bld[pl.ds(cnt + t * 16, 16)] = jnp.full((16,), LROWS, jnp.int32) + lane

    cnt = cnt + BS
    nfull = cnt // BS

    def flush2(b, _):
        pltpu.sync_copy(bsrc.at[pl.ds(b * BS, BS)],
                        lsrc_hbm.at[w, pl.ds((blk + b) * BS, BS)])
        pltpu.sync_copy(bld.at[pl.ds(b * BS, BS)],
                        lld_hbm.at[w, pl.ds((blk + b) * BS, BS)])
        return 0

    lax.fori_loop(0, nfull, flush2, 0)
    blk = blk + nfull

    cntv[...] = jnp.full((16,), 0, jnp.int32) + blk
    pltpu.sync_copy(cntv, cnt_hbm.at[w])

    @pl.loop(0, LROWS // 16)
    def _(i):
        acc = jnp.zeros((16,), jnp.float32)
        for j in range(16):
            acc = acc + hist[pl.ds(j * LACC + i * 16, 16)]
        degv[pl.ds(i * 16, 16)] = acc

    pltpu.sync_copy(degv, deg_hbm.at[pl.ds(lo, LROWS)])


@functools.partial(
    pl.kernel,
    out_type=jax.ShapeDtypeStruct((NPAD, H), jnp.float32),
    mesh=_MESH,
    compiler_params=pltpu.CompilerParams(needs_layout_passes=False),
    scratch_types=[
        pltpu.VMEM((2, HBS), jnp.int32),
        pltpu.VMEM((BS,), jnp.int32),
        pltpu.VMEM((16,), jnp.int32),
        pltpu.VMEM((2, HBS, H), jnp.float32),
        pltpu.VMEM((LACC, H), jnp.float32),
        pltpu.SemaphoreType.DMA,
        pltpu.SemaphoreType.DMA,
    ],
)
def _agg_kernel(lsrc_hbm, lld_hbm, cnt_hbm, y_hbm, out_hbm,
                sidx_v, ldv_v, cntv_v, rows_v, acc_v, sem_a, sem_b):
    c = lax.axis_index("c")
    s = lax.axis_index("s")
    w = s * 2 + c
    row0 = w * LROWS
    pltpu.sync_copy(y_hbm.at[pl.ds(row0, LROWS)], acc_v.at[pl.ds(0, LROWS)])
    pltpu.sync_copy(cnt_hbm.at[w], cntv_v)
    nblk = cntv_v[pl.ds(0, 16)][0]
    sems = (sem_a, sem_b)

    def start(b64, buf):
        pltpu.sync_copy(lsrc_hbm.at[w, pl.ds(b64 * HBS, HBS)],
                        sidx_v.at[buf])
        pltpu.async_copy(y_hbm.at[sidx_v.at[buf]], rows_v.at[buf], sems[buf])

    def finish(buf):
        pltpu.make_async_copy(y_hbm.at[sidx_v.at[buf]], rows_v.at[buf],
                              sems[buf]).wait()

    def accum(buf, half):
        @pl.loop(0, HBS // 16)
        def _(q):
            ld16 = ldv_v[pl.ds(half * HBS + q * 16, 16)]
            lds = [ld16[j] for j in range(16)]
            for k in range(H // 16):
                sl = pl.ds(k * 16, 16)
                for j in range(16):
                    plsc.addupdate(acc_v.at[lds[j], sl],
                                   rows_v[buf, q * 16 + j, sl])

    start(0, 0)

    @pl.loop(0, nblk)
    def _(b):
        pltpu.sync_copy(lld_hbm.at[w, pl.ds(b * BS, BS)], ldv_v)
        start(b * 2 + 1, 1)
        finish(0)
        accum(0, 0)

        @pl.when(b + 1 < nblk)
        def _():
            start(b * 2 + 2, 0)

        finish(1)
        accum(1, 1)

    pltpu.sync_copy(acc_v.at[pl.ds(0, LROWS)], out_hbm.at[pl.ds(row0, LROWS)])


def _dinv_col(deg_ref):
    return 1.0 / jnp.sqrt(deg_ref[...] + 1.0)


def _tc1_body(deg_ref, x_ref, w_ref, o_ref):
    dinv = _dinv_col(deg_ref)
    o_ref[...] = jnp.dot(x_ref[...], w_ref[...],
                         preferred_element_type=jnp.float32) * dinv


def _tc2_body(deg_ref, agg_ref, b_ref, w_ref, o_ref):
    dinv = _dinv_col(deg_ref)
    h = jnp.maximum(agg_ref[...] * dinv + b_ref[...], 0.0)
    o_ref[...] = jnp.dot(h, w_ref[...],
                         preferred_element_type=jnp.float32) * dinv


def _tc3_body(deg_ref, agg_ref, b_ref, batch_ref, wh_ref, bh_ref, o_ref,
              sums, cnts):
    i = pl.program_id(0)

    @pl.when(i == 0)
    def _():
        sums[...] = jnp.zeros_like(sums)
        cnts[...] = jnp.zeros_like(cnts)

    dinv = _dinv_col(deg_ref)
    h = jnp.maximum(agg_ref[...] * dinv + b_ref[...], 0.0)
    oh = (batch_ref[...] ==
          lax.broadcasted_iota(jnp.int32, (G, ROW_BLK), 0)).astype(jnp.float32)
    sums[...] += jnp.dot(oh, h, preferred_element_type=jnp.float32,
                         precision=lax.Precision.HIGHEST)
    cnts[...] += jnp.dot(oh, jnp.ones((ROW_BLK, H), jnp.float32),
                         preferred_element_type=jnp.float32,
                         precision=lax.Precision.HIGHEST)

    @pl.when(i == NROW_BLK - 1)
    def _():
        pooled = sums[...] / jnp.maximum(cnts[...], 1.0)
        o_ref[...] = jnp.dot(pooled, wh_ref[...],
                             preferred_element_type=jnp.float32,
                         precision=lax.Precision.HIGHEST) + bh_ref[...]


def _tc1(deg2d, x_pad, W):
    return pl.pallas_call(
        _tc1_body,
        grid=(NROW_BLK,),
        in_specs=[
            pl.BlockSpec((ROW_BLK, 1), lambda i: (i, 0)),
            pl.BlockSpec((ROW_BLK, D), lambda i: (i, 0)),
            pl.BlockSpec((D, H), lambda i: (0, 0)),
        ],
        out_specs=pl.BlockSpec((ROW_BLK, H), lambda i: (i, 0)),
        out_shape=jax.ShapeDtypeStruct((NPAD, H), jnp.float32),
    )(deg2d, x_pad, W)


def _tc2(deg2d, agg, b2d, W):
    return pl.pallas_call(
        _tc2_body,
        grid=(NROW_BLK,),
        in_specs=[
            pl.BlockSpec((ROW_BLK, 1), lambda i: (i, 0)),
            pl.BlockSpec((ROW_BLK, H), lambda i: (i, 0)),
            pl.BlockSpec((1, H), lambda i: (0, 0)),
            pl.BlockSpec((H, H), lambda i: (0, 0)),
        ],
        out_specs=pl.BlockSpec((ROW_BLK, H), lambda i: (i, 0)),
        out_shape=jax.ShapeDtypeStruct((NPAD, H), jnp.float32),
    )(deg2d, agg, b2d, W)


def _tc3(deg2d, agg, b2d, batch2d, Wh, bh2d):
    return pl.pallas_call(
        _tc3_body,
        grid=(NROW_BLK,),
        in_specs=[
            pl.BlockSpec((ROW_BLK, 1), lambda i: (i, 0)),
            pl.BlockSpec((ROW_BLK, H), lambda i: (i, 0)),
            pl.BlockSpec((1, H), lambda i: (0, 0)),
            pl.BlockSpec((1, ROW_BLK), lambda i: (0, i)),
            pl.BlockSpec((H, 1), lambda i: (0, 0)),
            pl.BlockSpec((1, 1), lambda i: (0, 0)),
        ],
        out_specs=pl.BlockSpec((G, 1), lambda i: (0, 0)),
        out_shape=jax.ShapeDtypeStruct((G, 1), jnp.float32),
        scratch_shapes=[
            pltpu.VMEM((G, H), jnp.float32),
            pltpu.VMEM((G, H), jnp.float32),
        ],
    )(deg2d, agg, b2d, batch2d, Wh, bh2d)


def kernel(x, edge_index, batch, W1, b1, W2, b2, Wh, bh):
    src = edge_index[0].astype(jnp.int32)
    dst = edge_index[1].astype(jnp.int32)
    pad_e = EPAD - E

    pad_ids = jnp.arange(pad_e, dtype=jnp.int32)
    src_pad = jnp.concatenate([src, (pad_ids * 97) % N])
    didx = jnp.concatenate([dst, N + pad_ids % (NPAD - N)])

    x_pad = jnp.pad(x, ((0, NPAD - N), (0, 0)))
    batch2d = jnp.pad(batch.astype(jnp.int32), (0, NPAD - N),
                      constant_values=G).reshape(1, NPAD)
    b1_2d = b1.reshape(1, H)
    b2_2d = b2.reshape(1, H)
    bh2d = bh.reshape(1, 1)

    deg, cnt16, lsrc, lld = _prep_kernel(didx, src_pad)
    deg2d = deg.reshape(NPAD, 1)

    y1 = _tc1(deg2d, x_pad, W1)
    agg1 = _agg_kernel(lsrc, lld, cnt16, y1)
    y2 = _tc2(deg2d, agg1, b1_2d, W2)
    agg2 = _agg_kernel(lsrc, lld, cnt16, y2)
    return _tc3(deg2d, agg2, b2_2d, batch2d, Wh, bh2d)
